# baseline (speedup 1.0000x reference)
import functools

import jax
import jax.numpy as jnp
from jax import lax
from jax.experimental import pallas as pl
from jax.experimental.pallas import tpu as pltpu
from jax.experimental.pallas import tpu_sc as plsc

BATCH = 1024
VOCAB = 1000
EMB = 64
VPAD = 1024
LANES = 16
NC = 2
NS = 16
NW = NC * NS
BPW = BATCH // NW


def _tc_main_body(cidx_ref, oidx_ref, wc_ref, wo_ref, es_ref, top_ref):
    col = lax.broadcasted_iota(jnp.int32, (BATCH, VPAD), 1)
    onehot_c = (col == cidx_ref[...]).astype(jnp.float32)
    onehot_o = (col == oidx_ref[...]).astype(jnp.float32)
    ce = jnp.dot(onehot_c, wc_ref[...],
                 preferred_element_type=jnp.float32)
    oe = jnp.dot(onehot_o, wo_ref[...],
                 preferred_element_type=jnp.float32)
    s = lax.dot_general(ce, wo_ref[...], (((1,), (1,)), ((), ())),
                        preferred_element_type=jnp.float32)
    es_ref[...] = jnp.where(col < VOCAB, jnp.exp(s), 0.0)
    top_ref[...] = jnp.sum(ce * oe, axis=1, keepdims=True)


_tc_main = pl.pallas_call(
    _tc_main_body,
    out_shape=[jax.ShapeDtypeStruct((BATCH, VPAD), jnp.float32),
               jax.ShapeDtypeStruct((BATCH, 1), jnp.float32)],
)


def _sc_lowsum_body(av_hbm, es_hbm, lp_hbm, av_v, es_v, lp_v, sem):
    wid = lax.axis_index("s") * NC + lax.axis_index("c")
    base = wid * BPW * VPAD
    c1 = pltpu.async_copy(av_hbm.at[pl.ds(base, BPW * VPAD)], av_v, sem)
    c2 = pltpu.async_copy(es_hbm.at[pl.ds(base, BPW * VPAD)], es_v, sem)
    c1.wait()
    c2.wait()

    def row_body(r, _):
        roff = r * VPAD

        def chunk_body(j, acc):
            a = av_v[pl.ds(roff + j * LANES, LANES)]
            g = plsc.load_gather(es_v, [a + roff])
            return acc + g

        acc = lax.fori_loop(0, VPAD // LANES, chunk_body,
                            jnp.zeros((LANES,), jnp.float32))
        lp_v[pl.ds(r * LANES, LANES)] = acc
        return 0

    lax.fori_loop(0, BPW, row_body, 0)
    pltpu.sync_copy(lp_v, lp_hbm.at[pl.ds(wid * BPW * LANES, BPW * LANES)])


_sc_lowsum = pl.kernel(
    _sc_lowsum_body,
    out_type=jax.ShapeDtypeStruct((BATCH * LANES,), jnp.float32),
    mesh=plsc.VectorSubcoreMesh(core_axis_name="c", subcore_axis_name="s"),
    scratch_types=[pltpu.VMEM((BPW * VPAD,), jnp.int32),
                   pltpu.VMEM((BPW * VPAD,), jnp.float32),
                   pltpu.VMEM((BPW * LANES,), jnp.float32),
                   pltpu.SemaphoreType.DMA],
    compiler_params=pltpu.CompilerParams(use_tc_tiling_on_sc=False,
                                         needs_layout_passes=False),
)


def _tc_final_body(lp_ref, top_ref, out_ref):
    low = jnp.sum(lp_ref[...], axis=1, keepdims=True)
    val = jnp.log(low) - top_ref[...]
    out_ref[...] = (jnp.sum(val) / BATCH).reshape(1, 1)


_tc_final = pl.pallas_call(
    _tc_final_body,
    out_shape=jax.ShapeDtypeStruct((1, 1), jnp.float32),
)


def kernel(center, outside, all_vocabs, W_center, W_outside):
    av_pad = jnp.pad(all_vocabs, ((0, 0), (0, VPAD - VOCAB)),
                     constant_values=VOCAB)
    wc_pad = jnp.pad(W_center, ((0, VPAD - VOCAB), (0, 0)))
    wo_pad = jnp.pad(W_outside, ((0, VPAD - VOCAB), (0, 0)))
    es, top = _tc_main(center, outside, wc_pad, wo_pad)
    lp = _sc_lowsum(av_pad.reshape(BATCH * VPAD),
                    es.reshape(BATCH * VPAD))
    loss = _tc_final(lp.reshape(BATCH, LANES), top)
    return loss[0, 0]


# speedup vs baseline: 1.2889x; 1.2889x over previous
import jax
import jax.numpy as jnp
from jax import lax
from jax.experimental import pallas as pl
from jax.experimental.pallas import tpu as pltpu
from jax.experimental.pallas import tpu_sc as plsc

BATCH = 1024
VOCAB = 1000
EMB = 64
VPAD = 1024
NCB = VPAD // 128
LANES = 16
NC = 2
NS = 16
NW = NC * NS
BPW = BATCH // NW


def _tc_main_body(cidx_ref, oidx_ref, wc_ref, wo_ref, av_ref,
                  es_ref, av3_ref, top_ref):
    col = lax.broadcasted_iota(jnp.int32, (BATCH, VPAD), 1)
    onehot_c = (col == cidx_ref[...]).astype(jnp.float32)
    onehot_o = (col == oidx_ref[...]).astype(jnp.float32)
    ce = jnp.dot(onehot_c, wc_ref[...],
                 preferred_element_type=jnp.float32)
    oe = jnp.dot(onehot_o, wo_ref[...],
                 preferred_element_type=jnp.float32)
    top_ref[...] = jnp.sum(ce * oe, axis=1, keepdims=True)
    av_pad = jnp.concatenate(
        [av_ref[...], jnp.full((BATCH, VPAD - VOCAB), VOCAB, jnp.int32)],
        axis=1)
    for cb in range(NCB):
        wo_cb = wo_ref[cb * 128:(cb + 1) * 128, :]
        s_cb = lax.dot_general(ce, wo_cb, (((1,), (1,)), ((), ())),
                               preferred_element_type=jnp.float32)
        if (cb + 1) * 128 > VOCAB:
            ccol = lax.broadcasted_iota(jnp.int32, (BATCH, 128), 1)
            es_cb = jnp.where(cb * 128 + ccol < VOCAB, jnp.exp(s_cb), 0.0)
        else:
            es_cb = jnp.exp(s_cb)
        es_ref[cb, :, :] = es_cb
        av3_ref[cb, :, :] = av_pad[:, cb * 128:(cb + 1) * 128]


_tc_main = pl.pallas_call(
    _tc_main_body,
    out_shape=[jax.ShapeDtypeStruct((NCB, BATCH, 128), jnp.float32),
               jax.ShapeDtypeStruct((NCB, BATCH, 128), jnp.int32),
               jax.ShapeDtypeStruct((BATCH, 1), jnp.float32)],
)


def _sc_lowsum_body(av_hbm, es_hbm, lp_hbm, av_v, es_v, lp_v, sem):
    wid = lax.axis_index("s") * NC + lax.axis_index("c")
    base = wid * BPW
    copies = []
    for cb in range(NCB):
        copies.append(pltpu.async_copy(av_hbm.at[cb, pl.ds(base, BPW)],
                                       av_v.at[cb], sem))
        copies.append(pltpu.async_copy(es_hbm.at[cb, pl.ds(base, BPW)],
                                       es_v.at[cb], sem))
    for c in copies:
        c.wait()

    def row_body(rloc, _):
        rvec = jnp.full((LANES,), rloc, jnp.int32)
        partial = []
        for cb in range(NCB):
            for j in range(128 // LANES):
                a = av_v[cb, rloc, pl.ds(j * LANES, LANES)]
                hi = lax.shift_right_logical(a, 7)
                lo = lax.bitwise_and(a, 127)
                partial.append(plsc.load_gather(es_v, [hi, rvec, lo]))
        while len(partial) > 1:
            partial = [partial[i] + partial[i + 1]
                       for i in range(0, len(partial) - 1, 2)] \
                      + ([partial[-1]] if len(partial) % 2 else [])
        lp_v[rloc >> 3, pl.ds((rloc & 7) * LANES, LANES)] = partial[0]
        return 0

    lax.fori_loop(0, BPW, row_body, 0)
    pltpu.sync_copy(lp_v, lp_hbm.at[pl.ds(wid * (BPW * LANES // 128),
                                          BPW * LANES // 128)])


_sc_lowsum = pl.kernel(
    _sc_lowsum_body,
    out_type=jax.ShapeDtypeStruct((BATCH * LANES // 128, 128), jnp.float32),
    mesh=plsc.VectorSubcoreMesh(core_axis_name="c", subcore_axis_name="s"),
    scratch_types=[pltpu.VMEM((NCB, BPW, 128), jnp.int32),
                   pltpu.VMEM((NCB, BPW, 128), jnp.float32),
                   pltpu.VMEM((BPW * LANES // 128, 128), jnp.float32),
                   pltpu.SemaphoreType.DMA],
    compiler_params=pltpu.CompilerParams(use_tc_tiling_on_sc=False,
                                         needs_layout_passes=False),
)


def _tc_final_body(lp_ref, top_ref, out_ref):
    cc = lax.broadcasted_iota(jnp.int32, (128, 128 // LANES), 0)
    gg = lax.broadcasted_iota(jnp.int32, (128, 128 // LANES), 1)
    sel = (lax.shift_right_logical(cc, 4) == gg).astype(jnp.float32)
    low = jnp.dot(lp_ref[...], sel,
                  preferred_element_type=jnp.float32)
    total = jnp.sum(jnp.log(low)) - jnp.sum(top_ref[...])
    out_ref[...] = (total / BATCH).reshape(1, 1)


_tc_final = pl.pallas_call(
    _tc_final_body,
    out_shape=jax.ShapeDtypeStruct((1, 1), jnp.float32),
)


def kernel(center, outside, all_vocabs, W_center, W_outside):
    wc_pad = jnp.pad(W_center, ((0, VPAD - VOCAB), (0, 0)))
    wo_pad = jnp.pad(W_outside, ((0, VPAD - VOCAB), (0, 0)))
    es, av3, top = _tc_main(center, outside, wc_pad, wo_pad, all_vocabs)
    lp = _sc_lowsum(av3, es)
    loss = _tc_final(lp, top)
    return loss[0, 0]
